# baseline (speedup 1.0000x reference)
import functools

import jax
import jax.numpy as jnp
from jax import lax
from jax.experimental import pallas as pl
from jax.experimental.pallas import tpu as pltpu
from jax.experimental.pallas import tpu_sc as plsc

B, S, D = 4, 4096, 1024
NC, NS = 2, 16
NW = NC * NS
ROWS_PER_W = S // NW
R = 16
CHUNKS = ROWS_PER_W // R
CHUNK = R * D

_mesh = plsc.VectorSubcoreMesh(core_axis_name="c", subcore_axis_name="s")


@functools.partial(
    pl.kernel,
    out_type=jax.ShapeDtypeStruct((B * S * D,), jnp.float32),
    mesh=_mesh,
    scratch_types=[
        pltpu.VMEM((CHUNK,), jnp.float32),
        pltpu.VMEM((CHUNK,), jnp.float32),
    ],
)
def _pos_add(in_hbm, emb_hbm, out_hbm, emb_v, buf_v):
    wid = lax.axis_index("s") * NC + lax.axis_index("c")
    base = wid * ROWS_PER_W * D
    for c in range(CHUNKS):
        off = base + c * CHUNK
        pltpu.sync_copy(emb_hbm.at[pl.ds(off, CHUNK)], emb_v)
        for b in range(B):
            pltpu.sync_copy(in_hbm.at[pl.ds(b * S * D + off, CHUNK)], buf_v)

            def add_body(i, carry):
                o = i * 16
                plsc.addupdate(buf_v.at[pl.ds(o, 16)], emb_v[pl.ds(o, 16)])
                return carry

            lax.fori_loop(0, CHUNK // 16, add_body, 0)
            pltpu.sync_copy(buf_v, out_hbm.at[pl.ds(b * S * D + off, CHUNK)])


def kernel(inputs, embedding):
    out = _pos_add(inputs.reshape(B * S * D), embedding.reshape(S * D))
    return out.reshape(B, S, D)


# speedup vs baseline: 1.5378x; 1.5378x over previous
import functools

import jax
import jax.numpy as jnp
from jax import lax
from jax.experimental import pallas as pl
from jax.experimental.pallas import tpu as pltpu
from jax.experimental.pallas import tpu_sc as plsc

B, S, D = 4, 4096, 1024
NC, NS = 2, 16
NW = NC * NS
ROWS_PER_W = S // NW
R = 16
CHUNKS = ROWS_PER_W // R
CHUNK = R * D
NBUF = 4
T = CHUNKS * B

_mesh = plsc.VectorSubcoreMesh(core_axis_name="c", subcore_axis_name="s")


@functools.partial(
    pl.kernel,
    out_type=jax.ShapeDtypeStruct((B * S * D,), jnp.float32),
    mesh=_mesh,
    scratch_types=[
        pltpu.VMEM((2, CHUNK), jnp.float32),
        pltpu.VMEM((NBUF, CHUNK), jnp.float32),
        pltpu.SemaphoreType.DMA((2,)),
        pltpu.SemaphoreType.DMA((NBUF,)),
        pltpu.SemaphoreType.DMA((NBUF,)),
    ],
)
def _pos_add(in_hbm, emb_hbm, out_hbm, emb_v, buf_v, emb_sem, in_sem, out_sem):
    wid = lax.axis_index("s") * NC + lax.axis_index("c")
    base = wid * ROWS_PER_W * D

    def emb_copy(c):
        return pltpu.make_async_copy(
            emb_hbm.at[pl.ds(base + c * CHUNK, CHUNK)],
            emb_v.at[c % 2], emb_sem.at[c % 2])

    def in_copy(t):
        c, b = divmod(t, B)
        return pltpu.make_async_copy(
            in_hbm.at[pl.ds(b * S * D + base + c * CHUNK, CHUNK)],
            buf_v.at[t % NBUF], in_sem.at[t % NBUF])

    def out_copy(t):
        c, b = divmod(t, B)
        return pltpu.make_async_copy(
            buf_v.at[t % NBUF],
            out_hbm.at[pl.ds(b * S * D + base + c * CHUNK, CHUNK)],
            out_sem.at[t % NBUF])

    emb_copy(0).start()
    in_copy(0).start()
    in_copy(1).start()

    for t in range(T):
        c, b = divmod(t, B)
        if b == 0:
            emb_copy(c).wait()
            if c + 1 < CHUNKS:
                emb_copy(c + 1).start()
        in_copy(t).wait()
        if t + 2 < T:
            if t - 2 >= 0:
                out_copy(t - 2).wait()
            in_copy(t + 2).start()

        buf = buf_v.at[t % NBUF]
        emb = emb_v.at[c % 2]

        @plsc.parallel_loop(0, CHUNK, step=16, unroll=8)
        def add_body(i):
            plsc.addupdate(buf.at[pl.ds(i, 16)], emb[pl.ds(i, 16)])

        out_copy(t).start()

    out_copy(T - 2).wait()
    out_copy(T - 1).wait()


def kernel(inputs, embedding):
    out = _pos_add(inputs.reshape(B * S * D), embedding.reshape(S * D))
    return out.reshape(B, S, D)


# speedup vs baseline: 4.8700x; 3.1667x over previous
import functools

import jax
import jax.numpy as jnp
from jax import lax
from jax.experimental import pallas as pl
from jax.experimental.pallas import tpu as pltpu
from jax.experimental.pallas import tpu_sc as plsc

B, S, D = 4, 4096, 1024
NC, NS = 2, 16
NW = NC * NS
ROWS_PER_W = S // NW
R = 16
CHUNKS = ROWS_PER_W // R
NBUF = 4
T = CHUNKS * B

_mesh = plsc.VectorSubcoreMesh(core_axis_name="c", subcore_axis_name="s")


@functools.partial(
    pl.kernel,
    out_type=jax.ShapeDtypeStruct((B, S, D), jnp.float32),
    mesh=_mesh,
    scratch_types=[
        pltpu.VMEM((2, R, D), jnp.float32),
        pltpu.VMEM((NBUF, R, D), jnp.float32),
        pltpu.SemaphoreType.DMA((2,)),
        pltpu.SemaphoreType.DMA((NBUF,)),
        pltpu.SemaphoreType.DMA((NBUF,)),
    ],
)
def _pos_add(in_hbm, emb_hbm, out_hbm, emb_v, buf_v, emb_sem, in_sem, out_sem):
    wid = lax.axis_index("s") * NC + lax.axis_index("c")
    row_base = wid * ROWS_PER_W

    def emb_copy(c):
        return pltpu.make_async_copy(
            emb_hbm.at[pl.ds(row_base + c * R, R)],
            emb_v.at[c % 2], emb_sem.at[c % 2])

    def in_copy(t):
        c, b = divmod(t, B)
        return pltpu.make_async_copy(
            in_hbm.at[b, pl.ds(row_base + c * R, R)],
            buf_v.at[t % NBUF], in_sem.at[t % NBUF])

    def out_copy(t):
        c, b = divmod(t, B)
        return pltpu.make_async_copy(
            buf_v.at[t % NBUF],
            out_hbm.at[b, pl.ds(row_base + c * R, R)],
            out_sem.at[t % NBUF])

    emb_copy(0).start()
    in_copy(0).start()
    in_copy(1).start()

    for t in range(T):
        c, b = divmod(t, B)
        if b == 0:
            emb_copy(c).wait()
            if c + 1 < CHUNKS:
                emb_copy(c + 1).start()
        in_copy(t).wait()
        if t + 2 < T:
            if t - 2 >= 0:
                out_copy(t - 2).wait()
            in_copy(t + 2).start()

        buf = buf_v.at[t % NBUF]
        emb = emb_v.at[c % 2]

        @plsc.parallel_loop(0, R * D, step=16, unroll=8)
        def add_body(i):
            r = i // D
            o = i % D
            plsc.addupdate(buf.at[r, pl.ds(o, 16)], emb[r, pl.ds(o, 16)])

        out_copy(t).start()

    out_copy(T - 2).wait()
    out_copy(T - 1).wait()


def kernel(inputs, embedding):
    return _pos_add(inputs, embedding)
